# baseline (speedup 1.0000x reference)
import functools

import jax
import jax.numpy as jnp
from jax import lax
from jax.experimental import pallas as pl
from jax.experimental.pallas import tpu as pltpu
from jax.experimental.pallas import tpu_sc as plsc

_N = 10000
_E = 320000
_NC = 2
_NS = 16
_NW = _NC * _NS
_EPW = _E // _NW
_K = 80
_NCHUNK = _EPW // _K
_STRIPE = 624
_TAIL = _N - _STRIPE * _NS

_mesh = plsc.VectorSubcoreMesh(core_axis_name="c", subcore_axis_name="s")


def _deg_body(src_hbm, dst_hbm, ew_hbm, out_hbm, dacc_o, dacc_i, zb,
              srcv, dstv1, dstv, ewv, semd):
    cid = lax.axis_index("c")
    sid = lax.axis_index("s")
    wid = sid * _NC + cid
    z16 = jnp.zeros((16,), jnp.float32)
    ebase = wid * _EPW

    d1 = pltpu.async_copy(src_hbm.at[pl.ds(ebase, _EPW)], srcv, semd)
    d2 = pltpu.async_copy(dst_hbm.at[pl.ds(ebase, _EPW)], dstv1, semd)
    d3 = pltpu.async_copy(ew_hbm.at[pl.ds(ebase, _EPW)], ewv, semd)

    def zb_body(i, _):
        zb[pl.ds(i * 16, 16)] = z16
        return 0
    lax.fori_loop(0, 40, zb_body, 0)

    base = sid * _STRIPE
    pltpu.sync_copy(zb.at[pl.ds(0, _STRIPE)], dacc_o.at[pl.ds(base, _STRIPE)])
    pltpu.sync_copy(zb.at[pl.ds(0, _STRIPE)], dacc_i.at[pl.ds(base, _STRIPE)])

    @pl.when(sid == _NS - 1)
    def _():
        tb = _STRIPE * _NS
        pltpu.sync_copy(zb.at[pl.ds(0, _TAIL)], dacc_o.at[pl.ds(tb, _TAIL)])
        pltpu.sync_copy(zb.at[pl.ds(0, _TAIL)], dacc_i.at[pl.ds(tb, _TAIL)])

    d1.wait()
    d2.wait()
    d3.wait()

    def repack(i, _):
        for g in range(_K // 16):
            dstv[i, pl.ds(g * 16, 16)] = dstv1[pl.ds(i * _K + g * 16, 16)]
        return 0
    lax.fori_loop(0, _NCHUNK, repack, 0)
    plsc.subcore_barrier()

    GRP = 5
    def group(g, _):
        descs = []
        for j in range(GRP):
            i = g * GRP + j
            descs.append(pltpu.async_copy(
                ewv.at[pl.ds(i * _K, _K)], dacc_o.at[srcv.at[pl.ds(i * _K, _K)]],
                semd, add=True))
            descs.append(pltpu.async_copy(
                ewv.at[pl.ds(i * _K, _K)], dacc_i.at[dstv.at[i]],
                semd, add=True))
        for d in descs:
            d.wait()
        return 0
    lax.fori_loop(0, _NCHUNK // GRP, group, 0)

    plsc.subcore_barrier()

    obase = cid * 2 * _N
    pltpu.sync_copy(dacc_o.at[pl.ds(base, _STRIPE)], zb.at[pl.ds(0, _STRIPE)])
    pltpu.sync_copy(zb.at[pl.ds(0, _STRIPE)],
                    out_hbm.at[pl.ds(obase + base, _STRIPE)])
    pltpu.sync_copy(dacc_i.at[pl.ds(base, _STRIPE)], zb.at[pl.ds(0, _STRIPE)])
    pltpu.sync_copy(zb.at[pl.ds(0, _STRIPE)],
                    out_hbm.at[pl.ds(obase + _N + base, _STRIPE)])

    @pl.when(sid == _NS - 1)
    def _():
        tb = _STRIPE * _NS
        pltpu.sync_copy(dacc_o.at[pl.ds(tb, _TAIL)], zb.at[pl.ds(0, _TAIL)])
        pltpu.sync_copy(zb.at[pl.ds(0, _TAIL)],
                        out_hbm.at[pl.ds(obase + tb, _TAIL)])
        pltpu.sync_copy(dacc_i.at[pl.ds(tb, _TAIL)], zb.at[pl.ds(0, _TAIL)])
        pltpu.sync_copy(zb.at[pl.ds(0, _TAIL)],
                        out_hbm.at[pl.ds(obase + _N + tb, _TAIL)])


_deg_call = functools.partial(
    pl.kernel,
    out_type=jax.ShapeDtypeStruct((_NC * 2 * _N,), jnp.float32),
    mesh=_mesh,
    scratch_types=[
        pltpu.VMEM_SHARED((_N,), jnp.float32),
        pltpu.VMEM_SHARED((_N,), jnp.float32),
        pltpu.VMEM((640,), jnp.float32),
        pltpu.VMEM((_EPW,), jnp.int32),
        pltpu.VMEM((_EPW,), jnp.int32),
        pltpu.VMEM((_NCHUNK, _K), jnp.int32),
        pltpu.VMEM((_EPW,), jnp.float32),
        pltpu.SemaphoreType.DMA,
    ],
)(_deg_body)


def _make_agg(H):
    ZR = 48
    NB = 3

    def body(h_hbm, src_hbm, dst_hbm, ew_hbm, out_hbm, acc, zb,
             src0, src1, src2, dst0, dst1, dst2, ew0, ew1, ew2,
             rows0, rows1, rows2,
             gsem0, gsem1, gsem2, isem0, isem1, isem2,
             ssem0, ssem1, ssem2):
        cid = lax.axis_index("c")
        sid = lax.axis_index("s")
        wid = sid * _NC + cid
        z16 = jnp.zeros((16,), jnp.float32)
        srcb = (src0, src1, src2)
        dstb = (dst0, dst1, dst2)
        ewb = (ew0, ew1, ew2)
        rows = (rows0, rows1, rows2)
        gsem = (gsem0, gsem1, gsem2)
        isem = (isem0, isem1, isem2)
        ssem = (ssem0, ssem1, ssem2)
        ebase = wid * _EPW

        def istart(i, b):
            off = ebase + i * _K
            pltpu.async_copy(src_hbm.at[pl.ds(off, _K)], srcb[b], isem[b])
            pltpu.async_copy(dst_hbm.at[pl.ds(off, _K)], dstb[b], isem[b])
            pltpu.async_copy(ew_hbm.at[pl.ds(off, _K)], ewb[b], isem[b])

        def iwait(i, b):
            off = ebase + i * _K
            pltpu.make_async_copy(src_hbm.at[pl.ds(off, _K)], srcb[b],
                                  isem[b]).wait()
            pltpu.make_async_copy(dst_hbm.at[pl.ds(off, _K)], dstb[b],
                                  isem[b]).wait()
            pltpu.make_async_copy(ew_hbm.at[pl.ds(off, _K)], ewb[b],
                                  isem[b]).wait()

        def gstart(i, b):
            pltpu.async_copy(h_hbm.at[srcb[b]], rows[b], gsem[b])

        def gwait(i, b):
            pltpu.make_async_copy(h_hbm.at[srcb[b]], rows[b], gsem[b]).wait()

        def sstart(i, b):
            pltpu.async_copy(rows[b], acc.at[dstb[b]], ssem[b], add=True)

        def swait(i, b):
            pltpu.make_async_copy(rows[b], acc.at[dstb[b]], ssem[b]).wait()

        def scale(i, b):
            rb = rows[b]
            eb = ewb[b]

            def sbody(g, _):
                ew16 = eb[pl.ds(g * 16, 16)]
                for l in range(16):
                    cv = jnp.full((16,), ew16[l], jnp.float32)
                    k = g * 16 + l
                    for j in range(H // 16):
                        rb[k, pl.ds(j * 16, 16)] = (
                            rb[k, pl.ds(j * 16, 16)] * cv)
                return 0
            lax.fori_loop(0, _K // 16, sbody, 0)

        istart(0, 0)
        istart(1, 1)

        def zb_body(i, _):
            for j in range(H // 16):
                zb[i, pl.ds(j * 16, 16)] = z16
            return 0
        lax.fori_loop(0, ZR, zb_body, 0)

        base = sid * _STRIPE
        for j in range(_STRIPE // ZR):
            pltpu.sync_copy(zb, acc.at[pl.ds(base + j * ZR, ZR)])

        @pl.when(sid == _NS - 1)
        def _():
            pltpu.sync_copy(zb.at[pl.ds(0, _TAIL)],
                            acc.at[pl.ds(_STRIPE * _NS, _TAIL)])

        plsc.subcore_barrier()

        iwait(0, 0)
        gstart(0, 0)
        iwait(1, 1)
        gstart(1, 1)
        gwait(0, 0)
        scale(0, 0)
        sstart(0, 0)
        istart(2, 2)

        def chunk_body(i, bcur, bnext, bprev):
            iwait(i + 1, bnext)
            gstart(i + 1, bnext)
            gwait(i, bcur)
            scale(i, bcur)
            sstart(i, bcur)
            swait(i - 1, bprev)

            @pl.when(i + 2 < _NCHUNK)
            def _():
                istart(i + 2, bprev)

        def triple(ii, _):
            i0 = 1 + 3 * ii
            chunk_body(i0, 1, 2, 0)
            chunk_body(i0 + 1, 2, 0, 1)
            chunk_body(i0 + 2, 0, 1, 2)
            return 0
        lax.fori_loop(0, (_NCHUNK - 2) // 3, triple, 0)

        last = _NCHUNK - 1
        gwait(last, 1)
        scale(last, 1)
        sstart(last, 1)
        swait(last - 1, 0)
        swait(last, 1)

        plsc.subcore_barrier()

        for j in range(_STRIPE // ZR):
            pltpu.sync_copy(acc.at[pl.ds(base + j * ZR, ZR)], zb)
            pltpu.sync_copy(zb, out_hbm.at[cid, pl.ds(base + j * ZR, ZR)])

        @pl.when(sid == _NS - 1)
        def _():
            pltpu.sync_copy(acc.at[pl.ds(_STRIPE * _NS, _TAIL)],
                            zb.at[pl.ds(0, _TAIL)])
            pltpu.sync_copy(zb.at[pl.ds(0, _TAIL)],
                            out_hbm.at[cid, pl.ds(_STRIPE * _NS, _TAIL)])

    params = None
    if H % 128 != 0:
        params = pltpu.CompilerParams(use_tc_tiling_on_sc=False)
    return functools.partial(
        pl.kernel,
        out_type=jax.ShapeDtypeStruct((_NC, _N, H), jnp.float32),
        mesh=_mesh,
        compiler_params=params,
        scratch_types=(
            [pltpu.VMEM_SHARED((_N, H), jnp.float32),
             pltpu.VMEM((ZR, H), jnp.float32)]
            + [pltpu.VMEM((_K,), jnp.int32)] * 6
            + [pltpu.VMEM((_K,), jnp.float32)] * 3
            + [pltpu.VMEM((_K, H), jnp.float32)] * 3
            + [pltpu.SemaphoreType.DMA] * 9
        ),
    )(body)


_agg128 = _make_agg(128)
_agg64 = _make_agg(64)
_agg48 = _make_agg(48)


_R = 2000


def _pre_body(x_ref, w1_ref, wres_ref, bres_ref, dp_ref,
              h1_ref, res_ref, ns_ref, nd_ref):
    x = x_ref[...]
    do = dp_ref[:, 0, 0] + dp_ref[:, 1, 0]
    di = dp_ref[:, 0, 1] + dp_ref[:, 1, 1]
    ns = jnp.where(do > 0, lax.rsqrt(jnp.maximum(do, 1e-12)), 0.0)
    nd = jnp.where(di > 0, lax.rsqrt(jnp.maximum(di, 1e-12)), 0.0)
    ns_ref[...] = ns[:, None]
    nd_ref[...] = nd[:, None]
    h1_ref[...] = jnp.dot(x, w1_ref[...],
                          preferred_element_type=jnp.float32) * ns[:, None]
    res_ref[...] = jnp.dot(x, wres_ref[...],
                           preferred_element_type=jnp.float32) + bres_ref[...]


def _tc_pre(x, W1, Wres, bres, degparts_t):
    grid = _N // _R
    return pl.pallas_call(
        _pre_body,
        grid=(grid,),
        in_specs=[
            pl.BlockSpec((_R, 128), lambda i: (i, 0)),
            pl.BlockSpec((128, 128), lambda i: (0, 0)),
            pl.BlockSpec((128, 40), lambda i: (0, 0)),
            pl.BlockSpec((40,), lambda i: (0,)),
            pl.BlockSpec((_R, _NC, 2), lambda i: (i, 0, 0)),
        ],
        out_specs=[
            pl.BlockSpec((_R, 128), lambda i: (i, 0)),
            pl.BlockSpec((_R, 40), lambda i: (i, 0)),
            pl.BlockSpec((_R, 1), lambda i: (i, 0)),
            pl.BlockSpec((_R, 1), lambda i: (i, 0)),
        ],
        out_shape=[
            jax.ShapeDtypeStruct((_N, 128), jnp.float32),
            jax.ShapeDtypeStruct((_N, 40), jnp.float32),
            jax.ShapeDtypeStruct((_N, 1), jnp.float32),
            jax.ShapeDtypeStruct((_N, 1), jnp.float32),
        ],
    )(x, W1, Wres, bres, degparts_t)


def _mid_body(p_ref, b_ref, nd_ref, w_ref, ns_ref, out_ref):
    p = p_ref[0] + p_ref[1]
    a = p * nd_ref[...] + b_ref[...]
    a = jnp.maximum(a, 0.0)
    out_ref[...] = jnp.dot(a, w_ref[...],
                           preferred_element_type=jnp.float32) * ns_ref[...]


def _tc_mid(part, b, nd, W, ns):
    H = part.shape[2]
    HO = W.shape[1]
    grid = _N // _R
    return pl.pallas_call(
        _mid_body,
        grid=(grid,),
        in_specs=[
            pl.BlockSpec((_NC, _R, H), lambda i: (0, i, 0)),
            pl.BlockSpec((H,), lambda i: (0,)),
            pl.BlockSpec((_R, 1), lambda i: (i, 0)),
            pl.BlockSpec((H, HO), lambda i: (0, 0)),
            pl.BlockSpec((_R, 1), lambda i: (i, 0)),
        ],
        out_specs=pl.BlockSpec((_R, HO), lambda i: (i, 0)),
        out_shape=jax.ShapeDtypeStruct((_N, HO), jnp.float32),
    )(part, b, nd, W, ns)


def _fin_body(p_ref, b_ref, nd_ref, res_ref, out_ref):
    p = p_ref[0] + p_ref[1]
    out_ref[...] = (p[:, :40] * nd_ref[...]
                    + b_ref[...] + res_ref[...])


def _tc_fin(part, b4, nd, res):
    grid = _N // _R
    return pl.pallas_call(
        _fin_body,
        grid=(grid,),
        in_specs=[
            pl.BlockSpec((_NC, _R, 48), lambda i: (0, i, 0)),
            pl.BlockSpec((40,), lambda i: (0,)),
            pl.BlockSpec((_R, 1), lambda i: (i, 0)),
            pl.BlockSpec((_R, 40), lambda i: (i, 0)),
        ],
        out_specs=pl.BlockSpec((_R, 40), lambda i: (i, 0)),
        out_shape=jax.ShapeDtypeStruct((_N, 40), jnp.float32),
    )(part, b4, nd, res)


def kernel(inputs, edge_index, edge_weight, W1, b1, W2, b2, W3, b3, W4, b4,
           Wres, bres):
    src = edge_index[0]
    dst = edge_index[1]
    ew = edge_weight
    W4p = jnp.pad(W4, ((0, 0), (0, 8)))

    degparts = _deg_call(src, dst, ew).reshape(_NC, 2, _N)
    degparts_t = jnp.transpose(degparts, (2, 0, 1))
    h1p, res, ns, nd = _tc_pre(inputs, W1, Wres, bres, degparts_t)
    p1 = _agg128(h1p, src, dst, ew)
    h2p = _tc_mid(p1, b1, nd, W2, ns)
    p2 = _agg128(h2p, src, dst, ew)
    h3p = _tc_mid(p2, b2, nd, W3, ns)
    p3 = _agg64(h3p, src, dst, ew)
    h4p = _tc_mid(p3, b3, nd, W4p, ns)
    p4 = _agg48(h4p, src, dst, ew)
    out = _tc_fin(p4, b4, nd, res)
    return out


# speedup vs baseline: 1.1479x; 1.1479x over previous
import functools

import jax
import jax.numpy as jnp
from jax import lax
from jax.experimental import pallas as pl
from jax.experimental.pallas import tpu as pltpu
from jax.experimental.pallas import tpu_sc as plsc

_N = 10000
_E = 320000
_NC = 2
_NS = 16
_NW = _NC * _NS
_EPW = _E // _NW
_K = 80
_NCHUNK = _EPW // _K
_STRIPE = 624
_TAIL = _N - _STRIPE * _NS

_mesh = plsc.VectorSubcoreMesh(core_axis_name="c", subcore_axis_name="s")


def _deg_body(src_hbm, dst_hbm, ew_hbm, out_hbm, dacc_o, dacc_i, zb,
              srcv, dstv1, dstv, ewv, semd):
    cid = lax.axis_index("c")
    sid = lax.axis_index("s")
    wid = sid * _NC + cid
    z16 = jnp.zeros((16,), jnp.float32)
    ebase = wid * _EPW

    d1 = pltpu.async_copy(src_hbm.at[pl.ds(ebase, _EPW)], srcv, semd)
    d2 = pltpu.async_copy(dst_hbm.at[pl.ds(ebase, _EPW)], dstv1, semd)
    d3 = pltpu.async_copy(ew_hbm.at[pl.ds(ebase, _EPW)], ewv, semd)

    def zb_body(i, _):
        zb[pl.ds(i * 16, 16)] = z16
        return 0
    lax.fori_loop(0, 40, zb_body, 0)

    base = sid * _STRIPE
    pltpu.sync_copy(zb.at[pl.ds(0, _STRIPE)], dacc_o.at[pl.ds(base, _STRIPE)])
    pltpu.sync_copy(zb.at[pl.ds(0, _STRIPE)], dacc_i.at[pl.ds(base, _STRIPE)])

    @pl.when(sid == _NS - 1)
    def _():
        tb = _STRIPE * _NS
        pltpu.sync_copy(zb.at[pl.ds(0, _TAIL)], dacc_o.at[pl.ds(tb, _TAIL)])
        pltpu.sync_copy(zb.at[pl.ds(0, _TAIL)], dacc_i.at[pl.ds(tb, _TAIL)])

    d1.wait()
    d2.wait()
    d3.wait()

    def repack(i, _):
        for g in range(_K // 16):
            dstv[i, pl.ds(g * 16, 16)] = dstv1[pl.ds(i * _K + g * 16, 16)]
        return 0
    lax.fori_loop(0, _NCHUNK, repack, 0)
    plsc.subcore_barrier()

    GRP = 5
    def group(g, _):
        descs = []
        for j in range(GRP):
            i = g * GRP + j
            descs.append(pltpu.async_copy(
                ewv.at[pl.ds(i * _K, _K)], dacc_o.at[srcv.at[pl.ds(i * _K, _K)]],
                semd, add=True))
            descs.append(pltpu.async_copy(
                ewv.at[pl.ds(i * _K, _K)], dacc_i.at[dstv.at[i]],
                semd, add=True))
        for d in descs:
            d.wait()
        return 0
    lax.fori_loop(0, _NCHUNK // GRP, group, 0)

    plsc.subcore_barrier()

    obase = cid * 2 * _N
    pltpu.sync_copy(dacc_o.at[pl.ds(base, _STRIPE)], zb.at[pl.ds(0, _STRIPE)])
    pltpu.sync_copy(zb.at[pl.ds(0, _STRIPE)],
                    out_hbm.at[pl.ds(obase + base, _STRIPE)])
    pltpu.sync_copy(dacc_i.at[pl.ds(base, _STRIPE)], zb.at[pl.ds(0, _STRIPE)])
    pltpu.sync_copy(zb.at[pl.ds(0, _STRIPE)],
                    out_hbm.at[pl.ds(obase + _N + base, _STRIPE)])

    @pl.when(sid == _NS - 1)
    def _():
        tb = _STRIPE * _NS
        pltpu.sync_copy(dacc_o.at[pl.ds(tb, _TAIL)], zb.at[pl.ds(0, _TAIL)])
        pltpu.sync_copy(zb.at[pl.ds(0, _TAIL)],
                        out_hbm.at[pl.ds(obase + tb, _TAIL)])
        pltpu.sync_copy(dacc_i.at[pl.ds(tb, _TAIL)], zb.at[pl.ds(0, _TAIL)])
        pltpu.sync_copy(zb.at[pl.ds(0, _TAIL)],
                        out_hbm.at[pl.ds(obase + _N + tb, _TAIL)])


_deg_call = functools.partial(
    pl.kernel,
    out_type=jax.ShapeDtypeStruct((_NC * 2 * _N,), jnp.float32),
    mesh=_mesh,
    scratch_types=[
        pltpu.VMEM_SHARED((_N,), jnp.float32),
        pltpu.VMEM_SHARED((_N,), jnp.float32),
        pltpu.VMEM((640,), jnp.float32),
        pltpu.VMEM((_EPW,), jnp.int32),
        pltpu.VMEM((_EPW,), jnp.int32),
        pltpu.VMEM((_NCHUNK, _K), jnp.int32),
        pltpu.VMEM((_EPW,), jnp.float32),
        pltpu.SemaphoreType.DMA,
    ],
)(_deg_body)


def _make_agg(H, tiled=True):
    ZR = 48
    NB = 3

    def body(h_hbm, src_hbm, dst_hbm, ew_hbm, out_hbm, acc, zb,
             src0, src1, src2, dst0, dst1, dst2, ew0, ew1, ew2,
             rows0, rows1, rows2,
             gsem0, gsem1, gsem2, isem0, isem1, isem2,
             ssem0, ssem1, ssem2):
        cid = lax.axis_index("c")
        sid = lax.axis_index("s")
        wid = sid * _NC + cid
        z16 = jnp.zeros((16,), jnp.float32)
        srcb = (src0, src1, src2)
        dstb = (dst0, dst1, dst2)
        ewb = (ew0, ew1, ew2)
        rows = (rows0, rows1, rows2)
        gsem = (gsem0, gsem1, gsem2)
        isem = (isem0, isem1, isem2)
        ssem = (ssem0, ssem1, ssem2)
        ebase = wid * _EPW

        def istart(i, b):
            off = ebase + i * _K
            pltpu.async_copy(src_hbm.at[pl.ds(off, _K)], srcb[b], isem[b])
            pltpu.async_copy(dst_hbm.at[pl.ds(off, _K)], dstb[b], isem[b])
            pltpu.async_copy(ew_hbm.at[pl.ds(off, _K)], ewb[b], isem[b])

        def iwait(i, b):
            off = ebase + i * _K
            pltpu.make_async_copy(src_hbm.at[pl.ds(off, _K)], srcb[b],
                                  isem[b]).wait()
            pltpu.make_async_copy(dst_hbm.at[pl.ds(off, _K)], dstb[b],
                                  isem[b]).wait()
            pltpu.make_async_copy(ew_hbm.at[pl.ds(off, _K)], ewb[b],
                                  isem[b]).wait()

        def gstart(i, b):
            pltpu.async_copy(h_hbm.at[srcb[b]], rows[b], gsem[b])

        def gwait(i, b):
            pltpu.make_async_copy(h_hbm.at[srcb[b]], rows[b], gsem[b]).wait()

        def sstart(i, b):
            pltpu.async_copy(rows[b], acc.at[dstb[b]], ssem[b], add=True)

        def swait(i, b):
            pltpu.make_async_copy(rows[b], acc.at[dstb[b]], ssem[b]).wait()

        def scale(i, b):
            rb = rows[b]
            eb = ewb[b]

            def sbody(g, _):
                ew16 = eb[pl.ds(g * 16, 16)]
                for l in range(16):
                    cv = jnp.full((16,), ew16[l], jnp.float32)
                    k = g * 16 + l
                    for j in range(H // 16):
                        rb[k, pl.ds(j * 16, 16)] = (
                            rb[k, pl.ds(j * 16, 16)] * cv)
                return 0
            lax.fori_loop(0, _K // 16, sbody, 0)

        istart(0, 0)
        istart(1, 1)

        def zb_body(i, _):
            for j in range(H // 16):
                zb[i, pl.ds(j * 16, 16)] = z16
            return 0
        lax.fori_loop(0, ZR, zb_body, 0)

        base = sid * _STRIPE
        for j in range(_STRIPE // ZR):
            pltpu.sync_copy(zb, acc.at[pl.ds(base + j * ZR, ZR)])

        @pl.when(sid == _NS - 1)
        def _():
            pltpu.sync_copy(zb.at[pl.ds(0, _TAIL)],
                            acc.at[pl.ds(_STRIPE * _NS, _TAIL)])

        plsc.subcore_barrier()

        iwait(0, 0)
        gstart(0, 0)
        iwait(1, 1)
        gstart(1, 1)
        gwait(0, 0)
        scale(0, 0)
        sstart(0, 0)
        istart(2, 2)

        def chunk_body(i, bcur, bnext, bprev):
            iwait(i + 1, bnext)
            gstart(i + 1, bnext)
            gwait(i, bcur)
            scale(i, bcur)
            sstart(i, bcur)
            swait(i - 1, bprev)

            @pl.when(i + 2 < _NCHUNK)
            def _():
                istart(i + 2, bprev)

        def triple(ii, _):
            i0 = 1 + 3 * ii
            chunk_body(i0, 1, 2, 0)
            chunk_body(i0 + 1, 2, 0, 1)
            chunk_body(i0 + 2, 0, 1, 2)
            return 0
        lax.fori_loop(0, (_NCHUNK - 2) // 3, triple, 0)

        last = _NCHUNK - 1
        gwait(last, 1)
        scale(last, 1)
        sstart(last, 1)
        swait(last - 1, 0)
        swait(last, 1)

        plsc.subcore_barrier()

        for j in range(_STRIPE // ZR):
            pltpu.sync_copy(acc.at[pl.ds(base + j * ZR, ZR)], zb)
            pltpu.sync_copy(zb, out_hbm.at[cid, pl.ds(base + j * ZR, ZR)])

        @pl.when(sid == _NS - 1)
        def _():
            pltpu.sync_copy(acc.at[pl.ds(_STRIPE * _NS, _TAIL)],
                            zb.at[pl.ds(0, _TAIL)])
            pltpu.sync_copy(zb.at[pl.ds(0, _TAIL)],
                            out_hbm.at[cid, pl.ds(_STRIPE * _NS, _TAIL)])

    params = None
    if not tiled:
        params = pltpu.CompilerParams(use_tc_tiling_on_sc=False)
    return functools.partial(
        pl.kernel,
        out_type=jax.ShapeDtypeStruct((_NC, _N, H), jnp.float32),
        mesh=_mesh,
        compiler_params=params,
        scratch_types=(
            [pltpu.VMEM_SHARED((_N, H), jnp.float32),
             pltpu.VMEM((ZR, H), jnp.float32)]
            + [pltpu.VMEM((_K,), jnp.int32)] * 6
            + [pltpu.VMEM((_K,), jnp.float32)] * 3
            + [pltpu.VMEM((_K, H), jnp.float32)] * 3
            + [pltpu.SemaphoreType.DMA] * 9
        ),
    )(body)


_agg128 = _make_agg(128)
_agg128u = _make_agg(128, tiled=False)
_agg64 = _make_agg(64, tiled=False)
_agg48 = _make_agg(48, tiled=False)


_R = 2000


def _pre_body(x_ref, w1_ref, wres_ref, bres_ref, dp_ref,
              h1_ref, res_ref, ns_ref, nd_ref):
    x = x_ref[...]
    do = dp_ref[:, 0, 0] + dp_ref[:, 1, 0]
    di = dp_ref[:, 0, 1] + dp_ref[:, 1, 1]
    ns = jnp.where(do > 0, lax.rsqrt(jnp.maximum(do, 1e-12)), 0.0)
    nd = jnp.where(di > 0, lax.rsqrt(jnp.maximum(di, 1e-12)), 0.0)
    ns_ref[...] = ns[:, None]
    nd_ref[...] = nd[:, None]
    h1_ref[...] = jnp.dot(x, w1_ref[...],
                          preferred_element_type=jnp.float32) * ns[:, None]
    res_ref[...] = jnp.dot(x, wres_ref[...],
                           preferred_element_type=jnp.float32) + bres_ref[...]


def _tc_pre(x, W1, Wres, bres, degparts_t):
    grid = _N // _R
    return pl.pallas_call(
        _pre_body,
        grid=(grid,),
        in_specs=[
            pl.BlockSpec((_R, 128), lambda i: (i, 0)),
            pl.BlockSpec((128, 128), lambda i: (0, 0)),
            pl.BlockSpec((128, 40), lambda i: (0, 0)),
            pl.BlockSpec((40,), lambda i: (0,)),
            pl.BlockSpec((_R, _NC, 2), lambda i: (i, 0, 0)),
        ],
        out_specs=[
            pl.BlockSpec((_R, 128), lambda i: (i, 0)),
            pl.BlockSpec((_R, 40), lambda i: (i, 0)),
            pl.BlockSpec((_R, 1), lambda i: (i, 0)),
            pl.BlockSpec((_R, 1), lambda i: (i, 0)),
        ],
        out_shape=[
            jax.ShapeDtypeStruct((_N, 128), jnp.float32),
            jax.ShapeDtypeStruct((_N, 40), jnp.float32),
            jax.ShapeDtypeStruct((_N, 1), jnp.float32),
            jax.ShapeDtypeStruct((_N, 1), jnp.float32),
        ],
    )(x, W1, Wres, bres, degparts_t)


def _mid_body(p_ref, b_ref, nd_ref, w_ref, ns_ref, out_ref):
    p = p_ref[0] + p_ref[1]
    a = p * nd_ref[...] + b_ref[...]
    a = jnp.maximum(a, 0.0)
    out_ref[...] = jnp.dot(a, w_ref[...],
                           preferred_element_type=jnp.float32) * ns_ref[...]


def _tc_mid(part, b, nd, W, ns):
    H = part.shape[2]
    HO = W.shape[1]
    grid = _N // _R
    return pl.pallas_call(
        _mid_body,
        grid=(grid,),
        in_specs=[
            pl.BlockSpec((_NC, _R, H), lambda i: (0, i, 0)),
            pl.BlockSpec((H,), lambda i: (0,)),
            pl.BlockSpec((_R, 1), lambda i: (i, 0)),
            pl.BlockSpec((H, HO), lambda i: (0, 0)),
            pl.BlockSpec((_R, 1), lambda i: (i, 0)),
        ],
        out_specs=pl.BlockSpec((_R, HO), lambda i: (i, 0)),
        out_shape=jax.ShapeDtypeStruct((_N, HO), jnp.float32),
    )(part, b, nd, W, ns)


def _fin_body(p_ref, b_ref, nd_ref, res_ref, out_ref):
    p = p_ref[0] + p_ref[1]
    out_ref[...] = (p[:, :40] * nd_ref[...]
                    + b_ref[...] + res_ref[...])


def _tc_fin(part, b4, nd, res):
    grid = _N // _R
    return pl.pallas_call(
        _fin_body,
        grid=(grid,),
        in_specs=[
            pl.BlockSpec((_NC, _R, 48), lambda i: (0, i, 0)),
            pl.BlockSpec((40,), lambda i: (0,)),
            pl.BlockSpec((_R, 1), lambda i: (i, 0)),
            pl.BlockSpec((_R, 40), lambda i: (i, 0)),
        ],
        out_specs=pl.BlockSpec((_R, 40), lambda i: (i, 0)),
        out_shape=jax.ShapeDtypeStruct((_N, 40), jnp.float32),
    )(part, b4, nd, res)


def kernel(inputs, edge_index, edge_weight, W1, b1, W2, b2, W3, b3, W4, b4,
           Wres, bres):
    src = edge_index[0]
    dst = edge_index[1]
    ew = edge_weight
    W4p = jnp.pad(W4, ((0, 0), (0, 8)))
    W3p = jnp.pad(W3, ((0, 0), (0, 64)))
    b3p = jnp.pad(b3, ((0, 64),))

    degparts = _deg_call(src, dst, ew).reshape(_NC, 2, _N)
    degparts_t = jnp.transpose(degparts, (2, 0, 1))
    h1p, res, ns, nd = _tc_pre(inputs, W1, Wres, bres, degparts_t)
    p1 = _agg128u(h1p, src, dst, ew)
    h2p = _tc_mid(p1, b1, nd, W2, ns)
    p2 = _agg128u(h2p, src, dst, ew)
    h3p = _tc_mid(p2, b2, nd, W3p, ns)
    p3 = _agg128u(h3p, src, dst, ew)
    h4p = _tc_mid(p3, b3p, nd, W4p, ns)
    p4 = _agg48(h4p, src, dst, ew)
    out = _tc_fin(p4, b4, nd, res)
    return out
